# baseline (speedup 1.0000x reference)
import functools

import jax
import jax.numpy as jnp
from jax import lax
from jax.experimental import pallas as pl
from jax.experimental.pallas import tpu as pltpu
from jax.experimental.pallas import tpu_sc as plsc

B, S, D, T = 16, 2048, 256, 32
NC, NS, L = 2, 16, 16
CHUNK = 128
HALF = S // 2
NCHUNK = HALF // CHUNK
ND = D // L
PAIRS = T * (T - 1) // 2
TPAD = T + L
PW = T * 4 * L


def _sload(ref, idx):
    return ref[pl.ds(idx, L)][0]


def _body(x_hbm, hi_hbm, lo_hbm, wc_hbm, b_hbm, out_hbm,
          buf0, buf1, hi_v, lo_v, wc_v, b_v, uv_v, lrow_v,
          seg_v, p16_v, pab_v, lsum_v, seg_sh, loss_sh, sem0, sem1):
    c = lax.axis_index("c")
    s = lax.axis_index("s")
    bl = s // 2
    half = s % 2
    batch = c * 8 + bl
    t0 = half * HALF
    row0 = batch * S + t0
    lanes = jnp.arange(L, dtype=jnp.int32)
    lane0 = lanes == 0
    zf = jnp.zeros((L,), jnp.float32)

    pltpu.sync_copy(hi_hbm.at[pl.ds(batch * T, T)], hi_v.at[pl.ds(0, T)])
    pltpu.sync_copy(lo_hbm.at[pl.ds(batch * T, T)], lo_v.at[pl.ds(0, T)])
    pltpu.sync_copy(wc_hbm, wc_v)
    pltpu.sync_copy(b_hbm, b_v)

    @pl.loop(0, T)
    def _z(t):
        for dd in range(ND):
            seg_v[pl.ds(t * D + dd * L, L)] = zf

    bufs = (buf0, buf1)
    sems = (sem0, sem1)
    pending = [None, None]
    pending[0] = pltpu.async_copy(x_hbm.at[pl.ds(row0, CHUNK)], buf0, sem0)
    for g in range(NCHUNK):
        pending[g % 2].wait()
        if g + 1 < NCHUNK:
            pending[(g + 1) % 2] = pltpu.async_copy(
                x_hbm.at[pl.ds(row0 + (g + 1) * CHUNK, CHUNK)],
                bufs[(g + 1) % 2], sems[(g + 1) % 2])
        base = t0 + g * CHUNK
        buf = bufs[g % 2]

        @pl.loop(0, T)
        def _seg(t):
            lot = jnp.maximum(_sload(lo_v, t), base)
            hit = jnp.minimum(_sload(hi_v, t), base + CHUNK)

            @pl.when(hit > lot)
            def _acc():
                def sbody(srl, acc):
                    r = srl - base
                    return tuple(a + buf[r, pl.ds(dd * L, L)]
                                 for dd, a in enumerate(acc))

                accs = lax.fori_loop(lot, hit, sbody,
                                     tuple(zf for _ in range(ND)))
                for dd in range(ND):
                    seg_v[pl.ds(t * D + dd * L, L)] = (
                        seg_v[pl.ds(t * D + dd * L, L)] + accs[dd])

    @pl.loop(0, T)
    def _proj(t):
        sv = [seg_v[pl.ds(t * D + dd * L, L)] for dd in range(ND)]
        for co in range(4):
            acc = sv[0] * wc_v[pl.ds(co * D, L)]
            for dd in range(1, ND):
                acc = acc + sv[dd] * wc_v[pl.ds(co * D + dd * L, L)]
            p16_v[pl.ds(t * (4 * L) + co * L, L)] = acc

    pltpu.sync_copy(p16_v, seg_sh.at[pl.ds(s * PW, PW)])
    plsc.subcore_barrier()

    @pl.when(half == 0)
    def _finalize():
        pltpu.sync_copy(seg_sh.at[pl.ds(s * PW, 2 * PW)], pab_v)

        @pl.loop(0, T)
        def _uv(t):
            cnt = _sload(hi_v, t) - _sload(lo_v, t)
            invn = 1.0 / (zf + cnt.astype(jnp.float32))
            for co in range(4):
                v = (pab_v[pl.ds(t * (4 * L) + co * L, L)]
                     + pab_v[pl.ds(PW + t * (4 * L) + co * L, L)])
                val = (zf + jnp.sum(v)) * invn
                plsc.store_scatter(uv_v, [jnp.full((L,), co * TPAD, jnp.int32) + t],
                                   val, mask=lane0)

        bvec = b_v[pl.ds(0, L)]
        b0 = bvec[0]
        b1 = bvec[1]

        def _cebody(j, acc):
            u0j = _sload(uv_v, j) + b0
            u1j = _sload(uv_v, TPAD + j) + b1
            tot = acc
            for gk in range(T // L):
                v0 = uv_v[pl.ds(2 * TPAD + gk * L, L)]
                v1 = uv_v[pl.ds(3 * TPAD + gk * L, L)]
                kv = lanes + (gk * L)
                l0 = v0 + u0j
                l1 = v1 + u1j
                m = jnp.maximum(l0, l1)
                e = jnp.exp(0.0 - jnp.abs(l0 - l1))
                r = e / (2.0 + e)
                r2 = r * r
                lny = 2.0 * r * (1.0 + r2 * (1.0 / 3.0 + r2 * (1.0 / 5.0 + r2 * (1.0 / 7.0 + r2 * (1.0 / 9.0)))))
                pick = jnp.where(kv == (j - 1), l1, l0) - (m + lny)
                tot = tot + jnp.where(kv < j, pick, jnp.zeros_like(pick))
            return tot

        lrow_v[...] = lax.fori_loop(0, T, _cebody, zf)

    @pl.when(half != 0)
    def _zl():
        lrow_v[...] = jnp.zeros((L,), jnp.float32)

    pltpu.sync_copy(lrow_v, loss_sh.at[pl.ds(s * L, L)])
    plsc.subcore_barrier()

    @pl.when(s == 0)
    def _final():
        pltpu.sync_copy(loss_sh, lsum_v)
        acc = lsum_v[pl.ds(0, L)]
        for se in range(1, NS):
            acc = acc + lsum_v[pl.ds(se * L, L)]
        total = jnp.sum(acc) * (-1.0 / (B * PAIRS))
        lrow_v[...] = jnp.zeros((L,), jnp.float32) + total
        pltpu.sync_copy(lrow_v, out_hbm.at[pl.ds(c * L, L)])


_sc_call = functools.partial(
    pl.kernel,
    out_type=jax.ShapeDtypeStruct((NC * L,), jnp.float32),
    mesh=plsc.VectorSubcoreMesh(core_axis_name="c", subcore_axis_name="s",
                                num_cores=NC, num_subcores=NS),
    compiler_params=pltpu.CompilerParams(use_tc_tiling_on_sc=True,
                                         needs_layout_passes=False),
    scratch_types=[
        pltpu.VMEM((CHUNK, D), jnp.float32),
        pltpu.VMEM((CHUNK, D), jnp.float32),
        pltpu.VMEM((TPAD,), jnp.int32),
        pltpu.VMEM((TPAD,), jnp.int32),
        pltpu.VMEM((4 * D,), jnp.float32),
        pltpu.VMEM((L,), jnp.float32),
        pltpu.VMEM((4 * TPAD,), jnp.float32),
        pltpu.VMEM((L,), jnp.float32),
        pltpu.VMEM((T * D,), jnp.float32),
        pltpu.VMEM((PW,), jnp.float32),
        pltpu.VMEM((2 * PW,), jnp.float32),
        pltpu.VMEM((NS * L,), jnp.float32),
        pltpu.VMEM_SHARED((NS * PW,), jnp.float32),
        pltpu.VMEM_SHARED((NS * L,), jnp.float32),
        pltpu.SemaphoreType.DMA,
        pltpu.SemaphoreType.DMA,
    ],
)(_body)


def kernel(encoder_output, his_turn_end_ids, W, b):
    ends = his_turn_end_ids.astype(jnp.int32)
    hi = (ends + 1).reshape(-1)
    lo = jnp.concatenate(
        [jnp.zeros((B, 1), jnp.int32), ends[:, :-1] + 1], axis=1).reshape(-1)
    wc = jnp.concatenate([W[:, :D], W[:, D:]], axis=0).reshape(-1)
    bpad = jnp.pad(b, (0, L - 2)).astype(jnp.float32)
    x = encoder_output.reshape(B * S, D)
    out = _sc_call(x, hi, lo, wc, bpad)
    return out[0] + out[L]


# speedup vs baseline: 1.0564x; 1.0564x over previous
import functools

import jax
import jax.numpy as jnp
from jax import lax
from jax.experimental import pallas as pl
from jax.experimental.pallas import tpu as pltpu
from jax.experimental.pallas import tpu_sc as plsc

B, S, D, T = 16, 2048, 256, 32
NC, NS, L = 2, 16, 16
CHUNK = 128
HALF = S // 2
NCHUNK = HALF // CHUNK
ND = D // L
PAIRS = T * (T - 1) // 2
TPAD = T + L
PW = T * 4 * L


def _sload(ref, idx):
    return ref[pl.ds(idx, L)][0]


def _body(x_hbm, hilo_hbm, wcb_hbm, out_hbm,
          buf0, buf1, hi_v, lo_v, wc_v, b_v, uv_v, lrow_v,
          seg_v, p16_v, pab_v, lsum_v, seg_sh, loss_sh, sem0, sem1):
    c = lax.axis_index("c")
    s = lax.axis_index("s")
    bl = s // 2
    half = s % 2
    batch = c * 8 + bl
    t0 = half * HALF
    lanes = jnp.arange(L, dtype=jnp.int32)
    lane0 = lanes == 0
    zf = jnp.zeros((L,), jnp.float32)

    pltpu.sync_copy(hilo_hbm.at[pl.ds(batch * T, T)], hi_v.at[pl.ds(0, T)])
    pltpu.sync_copy(hilo_hbm.at[pl.ds(B * T + batch * T, T)], lo_v.at[pl.ds(0, T)])
    pltpu.sync_copy(wcb_hbm.at[pl.ds(0, 4 * D)], wc_v)
    pltpu.sync_copy(wcb_hbm.at[pl.ds(4 * D, L)], b_v)

    def _count_lt(ref, bound):
        n = jnp.zeros((L,), jnp.int32)
        onei = jnp.ones((L,), jnp.int32)
        zeroi = jnp.zeros((L,), jnp.int32)
        for gk in range(T // L):
            n = n + jnp.where(ref[pl.ds(gk * L, L)] < bound, onei, zeroi)
        return jnp.sum(n)

    @pl.loop(0, T)
    def _z(t):
        for dd in range(ND):
            seg_v[pl.ds(t * D + dd * L, L)] = zf

    bufs = (buf0, buf1)
    sems = (sem0, sem1)
    pending = [None, None]
    pending[0] = pltpu.async_copy(x_hbm.at[batch, pl.ds(t0, CHUNK)], buf0, sem0)
    for g in range(NCHUNK):
        pending[g % 2].wait()
        if g + 1 < NCHUNK:
            pending[(g + 1) % 2] = pltpu.async_copy(
                x_hbm.at[batch, pl.ds(t0 + (g + 1) * CHUNK, CHUNK)],
                bufs[(g + 1) % 2], sems[(g + 1) % 2])
        base = t0 + g * CHUNK
        buf = bufs[g % 2]
        tlo = _count_lt(hi_v, base + 1)
        thi = _count_lt(lo_v, base + CHUNK)

        @pl.loop(tlo, thi)
        def _seg(t):
            lot = jnp.maximum(_sload(lo_v, t), base)
            hit = jnp.minimum(_sload(hi_v, t), base + CHUNK)

            @pl.when(hit > lot)
            def _acc():
                def sbody(srl, acc):
                    r = srl - base
                    return tuple(a + buf[r, pl.ds(dd * L, L)]
                                 for dd, a in enumerate(acc))

                accs = lax.fori_loop(lot, hit, sbody,
                                     tuple(zf for _ in range(ND)))
                for dd in range(ND):
                    seg_v[pl.ds(t * D + dd * L, L)] = (
                        seg_v[pl.ds(t * D + dd * L, L)] + accs[dd])

    @pl.loop(0, T)
    def _pz(t):
        for co in range(4):
            p16_v[pl.ds(t * (4 * L) + co * L, L)] = zf

    wlo = _count_lt(hi_v, t0 + 1)
    whi = _count_lt(lo_v, t0 + HALF)

    @pl.loop(wlo, whi)
    def _proj(t):
        sv = [seg_v[pl.ds(t * D + dd * L, L)] for dd in range(ND)]
        for co in range(4):
            acc = sv[0] * wc_v[pl.ds(co * D, L)]
            for dd in range(1, ND):
                acc = acc + sv[dd] * wc_v[pl.ds(co * D + dd * L, L)]
            p16_v[pl.ds(t * (4 * L) + co * L, L)] = acc

    pltpu.sync_copy(p16_v, seg_sh.at[pl.ds(s * PW, PW)])
    plsc.subcore_barrier()

    @pl.when(half == 0)
    def _finalize():
        pltpu.sync_copy(seg_sh.at[pl.ds(s * PW, 2 * PW)], pab_v)

        @pl.loop(0, T)
        def _uv(t):
            cnt = _sload(hi_v, t) - _sload(lo_v, t)
            invn = 1.0 / (zf + cnt.astype(jnp.float32))
            for co in range(4):
                v = (pab_v[pl.ds(t * (4 * L) + co * L, L)]
                     + pab_v[pl.ds(PW + t * (4 * L) + co * L, L)])
                val = (zf + jnp.sum(v)) * invn
                plsc.store_scatter(uv_v, [jnp.full((L,), co * TPAD, jnp.int32) + t],
                                   val, mask=lane0)

        bvec = b_v[pl.ds(0, L)]
        b0 = bvec[0]
        b1 = bvec[1]

        def _cebody(j, acc):
            u0j = _sload(uv_v, j) + b0
            u1j = _sload(uv_v, TPAD + j) + b1
            tot = acc
            for gk in range(T // L):
                v0 = uv_v[pl.ds(2 * TPAD + gk * L, L)]
                v1 = uv_v[pl.ds(3 * TPAD + gk * L, L)]
                kv = lanes + (gk * L)
                l0 = v0 + u0j
                l1 = v1 + u1j
                m = jnp.maximum(l0, l1)
                e = jnp.exp(0.0 - jnp.abs(l0 - l1))
                r = e / (2.0 + e)
                r2 = r * r
                lny = 2.0 * r * (1.0 + r2 * (1.0 / 3.0 + r2 * (1.0 / 5.0 + r2 * (1.0 / 7.0 + r2 * (1.0 / 9.0)))))
                pick = jnp.where(kv == (j - 1), l1, l0) - (m + lny)
                tot = tot + jnp.where(kv < j, pick, jnp.zeros_like(pick))
            return tot

        lrow_v[...] = lax.fori_loop(0, T, _cebody, zf)

    @pl.when(half != 0)
    def _zl():
        lrow_v[...] = jnp.zeros((L,), jnp.float32)

    pltpu.sync_copy(lrow_v, loss_sh.at[pl.ds(s * L, L)])
    plsc.subcore_barrier()

    @pl.when(s == 0)
    def _final():
        pltpu.sync_copy(loss_sh, lsum_v)
        acc = lsum_v[pl.ds(0, L)]
        for se in range(1, NS):
            acc = acc + lsum_v[pl.ds(se * L, L)]
        total = jnp.sum(acc) * (-1.0 / (B * PAIRS))
        lrow_v[...] = jnp.zeros((L,), jnp.float32) + total
        pltpu.sync_copy(lrow_v, out_hbm.at[pl.ds(c * L, L)])


_sc_call = functools.partial(
    pl.kernel,
    out_type=jax.ShapeDtypeStruct((NC * L,), jnp.float32),
    mesh=plsc.VectorSubcoreMesh(core_axis_name="c", subcore_axis_name="s",
                                num_cores=NC, num_subcores=NS),
    compiler_params=pltpu.CompilerParams(use_tc_tiling_on_sc=True,
                                         needs_layout_passes=False),
    scratch_types=[
        pltpu.VMEM((CHUNK, D), jnp.float32),
        pltpu.VMEM((CHUNK, D), jnp.float32),
        pltpu.VMEM((TPAD,), jnp.int32),
        pltpu.VMEM((TPAD,), jnp.int32),
        pltpu.VMEM((4 * D,), jnp.float32),
        pltpu.VMEM((L,), jnp.float32),
        pltpu.VMEM((4 * TPAD,), jnp.float32),
        pltpu.VMEM((L,), jnp.float32),
        pltpu.VMEM((T * D,), jnp.float32),
        pltpu.VMEM((PW,), jnp.float32),
        pltpu.VMEM((2 * PW,), jnp.float32),
        pltpu.VMEM((NS * L,), jnp.float32),
        pltpu.VMEM_SHARED((NS * PW,), jnp.float32),
        pltpu.VMEM_SHARED((NS * L,), jnp.float32),
        pltpu.SemaphoreType.DMA,
        pltpu.SemaphoreType.DMA,
    ],
)(_body)


def kernel(encoder_output, his_turn_end_ids, W, b):
    ends = his_turn_end_ids.astype(jnp.int32)
    hi = ends + 1
    lo = jnp.concatenate(
        [jnp.zeros((B, 1), jnp.int32), ends[:, :-1] + 1], axis=1)
    hilo = jnp.concatenate([hi.reshape(-1), lo.reshape(-1)])
    wcb = jnp.concatenate(
        [W[:, :D].reshape(-1)[: 2 * D], W[:, D:].reshape(-1),
         b.astype(jnp.float32), jnp.zeros((L - 2,), jnp.float32)])
    out = _sc_call(encoder_output, hilo, wcb)
    return out[0] + out[L]


# speedup vs baseline: 1.0628x; 1.0061x over previous
import functools

import jax
import jax.numpy as jnp
from jax import lax
from jax.experimental import pallas as pl
from jax.experimental.pallas import tpu as pltpu
from jax.experimental.pallas import tpu_sc as plsc

B, S, D, T = 16, 2048, 256, 32
NC, NS, L = 2, 16, 16
CHUNK = 128
HALF = S // 2
NCHUNK = HALF // CHUNK
ND = D // L
PAIRS = T * (T - 1) // 2
TPAD = T + L
PW = T * 4 * L


def _sload(ref, idx):
    return ref[pl.ds(idx, L)][0]


def _body(x_hbm, ends_hbm, w_hbm, b_hbm, out_hbm,
          buf0, buf1, ends_v, hi_v, lo_v, w2_v, b_v, uv_v, lrow_v,
          seg_v, p16_v, pab_v, lsum_v, seg_sh, loss_sh, sem0, sem1):
    c = lax.axis_index("c")
    s = lax.axis_index("s")
    bl = s // 2
    half = s % 2
    batch = c * 8 + bl
    t0 = half * HALF
    lanes = jnp.arange(L, dtype=jnp.int32)
    lane0 = lanes == 0
    zf = jnp.zeros((L,), jnp.float32)

    pltpu.sync_copy(ends_hbm.at[pl.ds(batch * T, T)], ends_v.at[pl.ds(0, T)])
    pltpu.sync_copy(w_hbm, w2_v)
    pltpu.sync_copy(b_hbm, b_v.at[pl.ds(0, 2)])

    onei = jnp.ones((L,), jnp.int32)
    hi0 = ends_v[pl.ds(0, L)] + onei
    hi1 = ends_v[pl.ds(L, L)] + onei
    hi_v[pl.ds(0, L)] = hi0
    hi_v[pl.ds(L, L)] = hi1
    lo_v[pl.ds(1, L)] = hi0
    lo_v[pl.ds(L + 1, L)] = hi1
    plsc.store_scatter(lo_v, [jnp.zeros((L,), jnp.int32)],
                       jnp.zeros((L,), jnp.int32), mask=(jnp.arange(L, dtype=jnp.int32) == 0))

    def _count_lt(ref, bound):
        n = jnp.zeros((L,), jnp.int32)
        onei = jnp.ones((L,), jnp.int32)
        zeroi = jnp.zeros((L,), jnp.int32)
        for gk in range(T // L):
            n = n + jnp.where(ref[pl.ds(gk * L, L)] < bound, onei, zeroi)
        return jnp.sum(n)

    @pl.loop(0, T)
    def _z(t):
        for dd in range(ND):
            seg_v[pl.ds(t * D + dd * L, L)] = zf

    bufs = (buf0, buf1)
    sems = (sem0, sem1)
    pending = [None, None]
    pending[0] = pltpu.async_copy(x_hbm.at[batch, pl.ds(t0, CHUNK)], buf0, sem0)
    for g in range(NCHUNK):
        pending[g % 2].wait()
        if g + 1 < NCHUNK:
            pending[(g + 1) % 2] = pltpu.async_copy(
                x_hbm.at[batch, pl.ds(t0 + (g + 1) * CHUNK, CHUNK)],
                bufs[(g + 1) % 2], sems[(g + 1) % 2])
        base = t0 + g * CHUNK
        buf = bufs[g % 2]
        tlo = _count_lt(hi_v, base + 1)
        thi = _count_lt(lo_v, base + CHUNK)

        @pl.loop(tlo, thi)
        def _seg(t):
            lot = jnp.maximum(_sload(lo_v, t), base)
            hit = jnp.minimum(_sload(hi_v, t), base + CHUNK)

            @pl.when(hit > lot)
            def _acc():
                def sbody(srl, acc):
                    r = srl - base
                    return tuple(a + buf[r, pl.ds(dd * L, L)]
                                 for dd, a in enumerate(acc))

                accs = lax.fori_loop(lot, hit, sbody,
                                     tuple(zf for _ in range(ND)))
                for dd in range(ND):
                    seg_v[pl.ds(t * D + dd * L, L)] = (
                        seg_v[pl.ds(t * D + dd * L, L)] + accs[dd])

    @pl.loop(0, T)
    def _pz(t):
        for co in range(4):
            p16_v[pl.ds(t * (4 * L) + co * L, L)] = zf

    wlo = _count_lt(hi_v, t0 + 1)
    whi = _count_lt(lo_v, t0 + HALF)

    @pl.loop(wlo, whi)
    def _proj(t):
        sv = [seg_v[pl.ds(t * D + dd * L, L)] for dd in range(ND)]
        for co in range(4):
            wrow, woff = co % 2, (co // 2) * D
            acc = sv[0] * w2_v[wrow, pl.ds(woff, L)]
            for dd in range(1, ND):
                acc = acc + sv[dd] * w2_v[wrow, pl.ds(woff + dd * L, L)]
            p16_v[pl.ds(t * (4 * L) + co * L, L)] = acc

    pltpu.sync_copy(p16_v, seg_sh.at[pl.ds(s * PW, PW)])
    plsc.subcore_barrier()

    @pl.when(half == 0)
    def _finalize():
        pltpu.sync_copy(seg_sh.at[pl.ds(s * PW, 2 * PW)], pab_v)

        @pl.loop(0, T)
        def _uv(t):
            cnt = _sload(hi_v, t) - _sload(lo_v, t)
            invn = 1.0 / (zf + cnt.astype(jnp.float32))
            for co in range(4):
                v = (pab_v[pl.ds(t * (4 * L) + co * L, L)]
                     + pab_v[pl.ds(PW + t * (4 * L) + co * L, L)])
                val = (zf + jnp.sum(v)) * invn
                plsc.store_scatter(uv_v, [jnp.full((L,), co * TPAD, jnp.int32) + t],
                                   val, mask=lane0)

        bvec = b_v[pl.ds(0, L)]
        b0 = bvec[0]
        b1 = bvec[1]

        def _cebody(j, acc):
            u0j = _sload(uv_v, j) + b0
            u1j = _sload(uv_v, TPAD + j) + b1
            tot = acc
            for gk in range(T // L):
                v0 = uv_v[pl.ds(2 * TPAD + gk * L, L)]
                v1 = uv_v[pl.ds(3 * TPAD + gk * L, L)]
                kv = lanes + (gk * L)
                l0 = v0 + u0j
                l1 = v1 + u1j
                m = jnp.maximum(l0, l1)
                e = jnp.exp(0.0 - jnp.abs(l0 - l1))
                r = e / (2.0 + e)
                r2 = r * r
                lny = 2.0 * r * (1.0 + r2 * (1.0 / 3.0 + r2 * (1.0 / 5.0 + r2 * (1.0 / 7.0 + r2 * (1.0 / 9.0)))))
                pick = jnp.where(kv == (j - 1), l1, l0) - (m + lny)
                tot = tot + jnp.where(kv < j, pick, jnp.zeros_like(pick))
            return tot

        lrow_v[...] = lax.fori_loop(0, T, _cebody, zf)

    @pl.when(half != 0)
    def _zl():
        lrow_v[...] = jnp.zeros((L,), jnp.float32)

    pltpu.sync_copy(lrow_v, loss_sh.at[pl.ds(s * L, L)])
    plsc.subcore_barrier()

    @pl.when(s == 0)
    def _final():
        pltpu.sync_copy(loss_sh, lsum_v)
        acc = lsum_v[pl.ds(0, L)]
        for se in range(1, NS):
            acc = acc + lsum_v[pl.ds(se * L, L)]
        total = jnp.sum(acc) * (-1.0 / (B * PAIRS))
        lrow_v[...] = jnp.zeros((L,), jnp.float32) + total
        pltpu.sync_copy(lrow_v, out_hbm.at[pl.ds(c * L, L)])


_sc_call = functools.partial(
    pl.kernel,
    out_type=jax.ShapeDtypeStruct((NC * L,), jnp.float32),
    mesh=plsc.VectorSubcoreMesh(core_axis_name="c", subcore_axis_name="s",
                                num_cores=NC, num_subcores=NS),
    compiler_params=pltpu.CompilerParams(use_tc_tiling_on_sc=True,
                                         needs_layout_passes=False),
    scratch_types=[
        pltpu.VMEM((CHUNK, D), jnp.float32),
        pltpu.VMEM((CHUNK, D), jnp.float32),
        pltpu.VMEM((TPAD,), jnp.int32),
        pltpu.VMEM((TPAD,), jnp.int32),
        pltpu.VMEM((TPAD,), jnp.int32),
        pltpu.VMEM((2, 2 * D), jnp.float32),
        pltpu.VMEM((L,), jnp.float32),
        pltpu.VMEM((4 * TPAD,), jnp.float32),
        pltpu.VMEM((L,), jnp.float32),
        pltpu.VMEM((T * D,), jnp.float32),
        pltpu.VMEM((PW,), jnp.float32),
        pltpu.VMEM((2 * PW,), jnp.float32),
        pltpu.VMEM((NS * L,), jnp.float32),
        pltpu.VMEM_SHARED((NS * PW,), jnp.float32),
        pltpu.VMEM_SHARED((NS * L,), jnp.float32),
        pltpu.SemaphoreType.DMA,
        pltpu.SemaphoreType.DMA,
    ],
)(_body)


def kernel(encoder_output, his_turn_end_ids, W, b):
    ends = his_turn_end_ids.astype(jnp.int32).reshape(-1)
    out = _sc_call(encoder_output, ends, W, b)
    return out[0] + out[L]


# speedup vs baseline: 1.1152x; 1.0493x over previous
import functools

import jax
import jax.numpy as jnp
from jax import lax
from jax.experimental import pallas as pl
from jax.experimental.pallas import tpu as pltpu
from jax.experimental.pallas import tpu_sc as plsc

B, S, D, T = 16, 2048, 256, 32
NC, NS, L = 2, 16, 16
CHUNK = 128
HALF = S // 2
NCHUNK = HALF // CHUNK
ND = D // L
PAIRS = T * (T - 1) // 2
TPAD = T + L
PW = T * 4 * L


def _sload(ref, idx):
    return ref[pl.ds(idx, L)][0]


def _body(x_hbm, ends_hbm, w_hbm, b_hbm, out_hbm,
          buf0, buf1, buf2, ends_v, hi_v, lo_v, w2_v, b_v, uv_v, lrow_v,
          seg_v, p16_v, pab_v, lsum_v, seg_sh, loss_sh, sem0, sem1, sem2):
    c = lax.axis_index("c")
    s = lax.axis_index("s")
    bl = s // 2
    half = s % 2
    batch = c * 8 + bl
    t0 = half * HALF
    lanes = jnp.arange(L, dtype=jnp.int32)
    lane0 = lanes == 0
    zf = jnp.zeros((L,), jnp.float32)

    pltpu.sync_copy(ends_hbm.at[pl.ds(batch * T, T)], ends_v.at[pl.ds(0, T)])
    pltpu.sync_copy(w_hbm, w2_v)
    pltpu.sync_copy(b_hbm, b_v.at[pl.ds(0, 2)])

    onei = jnp.ones((L,), jnp.int32)
    hi0 = ends_v[pl.ds(0, L)] + onei
    hi1 = ends_v[pl.ds(L, L)] + onei
    hi_v[pl.ds(0, L)] = hi0
    hi_v[pl.ds(L, L)] = hi1
    lo_v[pl.ds(1, L)] = hi0
    lo_v[pl.ds(L + 1, L)] = hi1
    plsc.store_scatter(lo_v, [jnp.zeros((L,), jnp.int32)],
                       jnp.zeros((L,), jnp.int32), mask=(jnp.arange(L, dtype=jnp.int32) == 0))

    def _count_lt(ref, bound):
        n = jnp.zeros((L,), jnp.int32)
        onei = jnp.ones((L,), jnp.int32)
        zeroi = jnp.zeros((L,), jnp.int32)
        for gk in range(T // L):
            n = n + jnp.where(ref[pl.ds(gk * L, L)] < bound, onei, zeroi)
        return jnp.sum(n)

    @pl.loop(0, T)
    def _z(t):
        for dd in range(ND):
            seg_v[pl.ds(t * D + dd * L, L)] = zf

    bufs = (buf0, buf1, buf2)
    sems = (sem0, sem1, sem2)
    NB = len(bufs)
    pending = [None] * NB
    for g0 in range(NB - 1):
        pending[g0] = pltpu.async_copy(
            x_hbm.at[batch, pl.ds(t0 + g0 * CHUNK, CHUNK)], bufs[g0], sems[g0])
    for g in range(NCHUNK):
        pending[g % NB].wait()
        if g + NB - 1 < NCHUNK:
            pending[(g + NB - 1) % NB] = pltpu.async_copy(
                x_hbm.at[batch, pl.ds(t0 + (g + NB - 1) * CHUNK, CHUNK)],
                bufs[(g + NB - 1) % NB], sems[(g + NB - 1) % NB])
        base = t0 + g * CHUNK
        buf = bufs[g % NB]
        tlo = _count_lt(hi_v, base + 1)
        thi = _count_lt(lo_v, base + CHUNK)

        @pl.loop(tlo, thi)
        def _seg(t):
            lot = jnp.maximum(_sload(lo_v, t), base)
            hit = jnp.minimum(_sload(hi_v, t), base + CHUNK)

            @pl.when(hit > lot)
            def _acc():
                def sbody(srl, acc):
                    r = srl - base
                    return tuple(a + buf[r, pl.ds(dd * L, L)]
                                 for dd, a in enumerate(acc))

                accs = lax.fori_loop(lot, hit, sbody,
                                     tuple(zf for _ in range(ND)))
                for dd in range(ND):
                    seg_v[pl.ds(t * D + dd * L, L)] = (
                        seg_v[pl.ds(t * D + dd * L, L)] + accs[dd])

    @pl.loop(0, T)
    def _pz(t):
        for co in range(4):
            p16_v[pl.ds(t * (4 * L) + co * L, L)] = zf

    wlo = _count_lt(hi_v, t0 + 1)
    whi = _count_lt(lo_v, t0 + HALF)

    @pl.loop(wlo, whi)
    def _proj(t):
        sv = [seg_v[pl.ds(t * D + dd * L, L)] for dd in range(ND)]
        for co in range(4):
            wrow, woff = co % 2, (co // 2) * D
            acc = sv[0] * w2_v[wrow, pl.ds(woff, L)]
            for dd in range(1, ND):
                acc = acc + sv[dd] * w2_v[wrow, pl.ds(woff + dd * L, L)]
            p16_v[pl.ds(t * (4 * L) + co * L, L)] = acc

    pltpu.sync_copy(p16_v, seg_sh.at[pl.ds(s * PW, PW)])
    plsc.subcore_barrier()

    @pl.when(half == 0)
    def _finalize():
        pltpu.sync_copy(seg_sh.at[pl.ds(s * PW, 2 * PW)], pab_v)

        @pl.loop(0, T)
        def _uv(t):
            cnt = _sload(hi_v, t) - _sload(lo_v, t)
            invn = 1.0 / (zf + cnt.astype(jnp.float32))
            for co in range(4):
                v = (pab_v[pl.ds(t * (4 * L) + co * L, L)]
                     + pab_v[pl.ds(PW + t * (4 * L) + co * L, L)])
                val = (zf + jnp.sum(v)) * invn
                plsc.store_scatter(uv_v, [jnp.full((L,), co * TPAD, jnp.int32) + t],
                                   val, mask=lane0)

        bvec = b_v[pl.ds(0, L)]
        b0 = bvec[0]
        b1 = bvec[1]

        def _cebody(j, acc):
            u0j = _sload(uv_v, j) + b0
            u1j = _sload(uv_v, TPAD + j) + b1
            tot = acc
            for gk in range(T // L):
                v0 = uv_v[pl.ds(2 * TPAD + gk * L, L)]
                v1 = uv_v[pl.ds(3 * TPAD + gk * L, L)]
                kv = lanes + (gk * L)
                l0 = v0 + u0j
                l1 = v1 + u1j
                m = jnp.maximum(l0, l1)
                e = jnp.exp(0.0 - jnp.abs(l0 - l1))
                r = e / (2.0 + e)
                r2 = r * r
                lny = 2.0 * r * (1.0 + r2 * (1.0 / 3.0 + r2 * (1.0 / 5.0 + r2 * (1.0 / 7.0 + r2 * (1.0 / 9.0)))))
                pick = jnp.where(kv == (j - 1), l1, l0) - (m + lny)
                tot = tot + jnp.where(kv < j, pick, jnp.zeros_like(pick))
            return tot

        lrow_v[...] = lax.fori_loop(0, T, _cebody, zf)

    @pl.when(half != 0)
    def _zl():
        lrow_v[...] = jnp.zeros((L,), jnp.float32)

    pltpu.sync_copy(lrow_v, loss_sh.at[pl.ds(s * L, L)])
    plsc.subcore_barrier()

    @pl.when(s == 0)
    def _final():
        pltpu.sync_copy(loss_sh, lsum_v)
        acc = lsum_v[pl.ds(0, L)]
        for se in range(1, NS):
            acc = acc + lsum_v[pl.ds(se * L, L)]
        total = jnp.sum(acc) * (-1.0 / (B * PAIRS))
        lrow_v[...] = jnp.zeros((L,), jnp.float32) + total
        pltpu.sync_copy(lrow_v, out_hbm.at[pl.ds(c * L, L)])


_sc_call = functools.partial(
    pl.kernel,
    out_type=jax.ShapeDtypeStruct((NC * L,), jnp.float32),
    mesh=plsc.VectorSubcoreMesh(core_axis_name="c", subcore_axis_name="s",
                                num_cores=NC, num_subcores=NS),
    compiler_params=pltpu.CompilerParams(use_tc_tiling_on_sc=True,
                                         needs_layout_passes=False),
    scratch_types=[
        pltpu.VMEM((CHUNK, D), jnp.float32),
        pltpu.VMEM((CHUNK, D), jnp.float32),
        pltpu.VMEM((CHUNK, D), jnp.float32),
        pltpu.VMEM((TPAD,), jnp.int32),
        pltpu.VMEM((TPAD,), jnp.int32),
        pltpu.VMEM((TPAD,), jnp.int32),
        pltpu.VMEM((2, 2 * D), jnp.float32),
        pltpu.VMEM((L,), jnp.float32),
        pltpu.VMEM((4 * TPAD,), jnp.float32),
        pltpu.VMEM((L,), jnp.float32),
        pltpu.VMEM((T * D,), jnp.float32),
        pltpu.VMEM((PW,), jnp.float32),
        pltpu.VMEM((2 * PW,), jnp.float32),
        pltpu.VMEM((NS * L,), jnp.float32),
        pltpu.VMEM_SHARED((NS * PW,), jnp.float32),
        pltpu.VMEM_SHARED((NS * L,), jnp.float32),
        pltpu.SemaphoreType.DMA,
        pltpu.SemaphoreType.DMA,
        pltpu.SemaphoreType.DMA,
    ],
)(_body)


def kernel(encoder_output, his_turn_end_ids, W, b):
    ends = his_turn_end_ids.astype(jnp.int32).reshape(-1)
    out = _sc_call(encoder_output, ends, W, b)
    return out[0] + out[L]


# speedup vs baseline: 1.1435x; 1.0253x over previous
import functools

import jax
import jax.numpy as jnp
from jax import lax
from jax.experimental import pallas as pl
from jax.experimental.pallas import tpu as pltpu
from jax.experimental.pallas import tpu_sc as plsc

B, S, D, T = 16, 2048, 256, 32
NC, NS, L = 2, 16, 16
CHUNK = 128
HALF = S // 2
NCHUNK = HALF // CHUNK
ND = D // L
PAIRS = T * (T - 1) // 2
TPAD = T + L
PW = T * 4 * L


def _sload(ref, idx):
    return ref[pl.ds(idx, L)][0]


def _body(x_hbm, ends_hbm, w_hbm, b_hbm, out_hbm,
          buf0, buf1, buf2, ends_v, hi_v, lo_v, w2_v, b_v, uv_v, lrow_v,
          seg_v, p16_v, pab_v, lsum_v, seg_sh, loss_sh, sem0, sem1, sem2):
    c = lax.axis_index("c")
    s = lax.axis_index("s")
    bl = s // 2
    half = s % 2
    batch = c * 8 + bl
    t0 = half * HALF
    lanes = jnp.arange(L, dtype=jnp.int32)
    lane0 = lanes == 0
    zf = jnp.zeros((L,), jnp.float32)

    bufs = (buf0, buf1, buf2)
    sems = (sem0, sem1, sem2)
    NB = len(bufs)
    pending = [None] * NB
    for g0 in range(NB - 1):
        pending[g0] = pltpu.async_copy(
            x_hbm.at[batch, pl.ds(t0 + g0 * CHUNK, CHUNK)], bufs[g0], sems[g0])

    pltpu.sync_copy(ends_hbm.at[pl.ds(batch * T, T)], ends_v.at[pl.ds(0, T)])
    pltpu.sync_copy(w_hbm, w2_v)
    pltpu.sync_copy(b_hbm, b_v.at[pl.ds(0, 2)])

    onei = jnp.ones((L,), jnp.int32)
    hi0 = ends_v[pl.ds(0, L)] + onei
    hi1 = ends_v[pl.ds(L, L)] + onei
    hi_v[pl.ds(0, L)] = hi0
    hi_v[pl.ds(L, L)] = hi1
    lo_v[pl.ds(1, L)] = hi0
    lo_v[pl.ds(L + 1, L)] = hi1
    plsc.store_scatter(lo_v, [jnp.zeros((L,), jnp.int32)],
                       jnp.zeros((L,), jnp.int32), mask=(jnp.arange(L, dtype=jnp.int32) == 0))

    def _count_lt(ref, bound):
        n = jnp.zeros((L,), jnp.int32)
        onei = jnp.ones((L,), jnp.int32)
        zeroi = jnp.zeros((L,), jnp.int32)
        for gk in range(T // L):
            n = n + jnp.where(ref[pl.ds(gk * L, L)] < bound, onei, zeroi)
        return jnp.sum(n)

    @pl.loop(0, T)
    def _z(t):
        for dd in range(ND):
            seg_v[pl.ds(t * D + dd * L, L)] = zf

    for g in range(NCHUNK):
        pending[g % NB].wait()
        if g + NB - 1 < NCHUNK:
            pending[(g + NB - 1) % NB] = pltpu.async_copy(
                x_hbm.at[batch, pl.ds(t0 + (g + NB - 1) * CHUNK, CHUNK)],
                bufs[(g + NB - 1) % NB], sems[(g + NB - 1) % NB])
        base = t0 + g * CHUNK
        buf = bufs[g % NB]
        tlo = _count_lt(hi_v, base + 1)
        thi = _count_lt(lo_v, base + CHUNK)

        @pl.loop(tlo, thi)
        def _seg(t):
            lot = jnp.maximum(_sload(lo_v, t), base)
            hit = jnp.minimum(_sload(hi_v, t), base + CHUNK)

            @pl.when(hit > lot)
            def _acc():
                def sbody(srl, acc):
                    r = srl - base
                    return tuple(a + buf[r, pl.ds(dd * L, L)]
                                 for dd, a in enumerate(acc))

                accs = lax.fori_loop(lot, hit, sbody,
                                     tuple(zf for _ in range(ND)))
                for dd in range(ND):
                    seg_v[pl.ds(t * D + dd * L, L)] = (
                        seg_v[pl.ds(t * D + dd * L, L)] + accs[dd])

    @pl.loop(0, T)
    def _pz(t):
        for co in range(4):
            p16_v[pl.ds(t * (4 * L) + co * L, L)] = zf

    wlo = _count_lt(hi_v, t0 + 1)
    whi = _count_lt(lo_v, t0 + HALF)

    @pl.loop(wlo, whi)
    def _proj(t):
        sv = [seg_v[pl.ds(t * D + dd * L, L)] for dd in range(ND)]
        for co in range(4):
            wrow, woff = co % 2, (co // 2) * D
            acc = sv[0] * w2_v[wrow, pl.ds(woff, L)]
            for dd in range(1, ND):
                acc = acc + sv[dd] * w2_v[wrow, pl.ds(woff + dd * L, L)]
            p16_v[pl.ds(t * (4 * L) + co * L, L)] = acc

    pltpu.sync_copy(p16_v, seg_sh.at[pl.ds(s * PW, PW)])
    plsc.subcore_barrier()

    pltpu.sync_copy(seg_sh.at[pl.ds(bl * 2 * PW, 2 * PW)], pab_v)

    if True:

        @pl.loop(0, T)
        def _uv(t):
            cnt = _sload(hi_v, t) - _sload(lo_v, t)
            invn = 1.0 / (zf + cnt.astype(jnp.float32))
            for co in range(4):
                v = (pab_v[pl.ds(t * (4 * L) + co * L, L)]
                     + pab_v[pl.ds(PW + t * (4 * L) + co * L, L)])
                val = (zf + jnp.sum(v)) * invn
                plsc.store_scatter(uv_v, [jnp.full((L,), co * TPAD, jnp.int32) + t],
                                   val, mask=lane0)

        bvec = b_v[pl.ds(0, L)]
        b0 = bvec[0]
        b1 = bvec[1]

        def _cebody(j, acc):
            u0j = _sload(uv_v, j) + b0
            u1j = _sload(uv_v, TPAD + j) + b1
            tot = acc
            for gk in range(T // L):
                v0 = uv_v[pl.ds(2 * TPAD + gk * L, L)]
                v1 = uv_v[pl.ds(3 * TPAD + gk * L, L)]
                kv = lanes + (gk * L)
                l0 = v0 + u0j
                l1 = v1 + u1j
                m = jnp.maximum(l0, l1)
                e = jnp.exp(0.0 - jnp.abs(l0 - l1))
                r = e / (2.0 + e)
                r2 = r * r
                lny = 2.0 * r * (1.0 + r2 * (1.0 / 3.0 + r2 * (1.0 / 5.0 + r2 * (1.0 / 7.0 + r2 * (1.0 / 9.0)))))
                pick = jnp.where(kv == (j - 1), l1, l0) - (m + lny)
                tot = tot + jnp.where(kv < j, pick, jnp.zeros_like(pick))
            return tot

        jlo = half * (T // 2)
        lrow_v[...] = lax.fori_loop(jlo, jlo + T // 2, _cebody, zf)

    pltpu.sync_copy(lrow_v, loss_sh.at[pl.ds(s * L, L)])
    plsc.subcore_barrier()

    @pl.when(s == 0)
    def _final():
        pltpu.sync_copy(loss_sh, lsum_v)
        acc = lsum_v[pl.ds(0, L)]
        for se in range(1, NS):
            acc = acc + lsum_v[pl.ds(se * L, L)]
        total = jnp.sum(acc) * (-1.0 / (B * PAIRS))
        lrow_v[...] = jnp.zeros((L,), jnp.float32) + total
        pltpu.sync_copy(lrow_v, out_hbm.at[pl.ds(c * L, L)])


_sc_call = functools.partial(
    pl.kernel,
    out_type=jax.ShapeDtypeStruct((NC * L,), jnp.float32),
    mesh=plsc.VectorSubcoreMesh(core_axis_name="c", subcore_axis_name="s",
                                num_cores=NC, num_subcores=NS),
    compiler_params=pltpu.CompilerParams(use_tc_tiling_on_sc=True,
                                         needs_layout_passes=False),
    scratch_types=[
        pltpu.VMEM((CHUNK, D), jnp.float32),
        pltpu.VMEM((CHUNK, D), jnp.float32),
        pltpu.VMEM((CHUNK, D), jnp.float32),
        pltpu.VMEM((TPAD,), jnp.int32),
        pltpu.VMEM((TPAD,), jnp.int32),
        pltpu.VMEM((TPAD,), jnp.int32),
        pltpu.VMEM((2, 2 * D), jnp.float32),
        pltpu.VMEM((L,), jnp.float32),
        pltpu.VMEM((4 * TPAD,), jnp.float32),
        pltpu.VMEM((L,), jnp.float32),
        pltpu.VMEM((T * D,), jnp.float32),
        pltpu.VMEM((PW,), jnp.float32),
        pltpu.VMEM((2 * PW,), jnp.float32),
        pltpu.VMEM((NS * L,), jnp.float32),
        pltpu.VMEM_SHARED((NS * PW,), jnp.float32),
        pltpu.VMEM_SHARED((NS * L,), jnp.float32),
        pltpu.SemaphoreType.DMA,
        pltpu.SemaphoreType.DMA,
        pltpu.SemaphoreType.DMA,
    ],
)(_body)


def kernel(encoder_output, his_turn_end_ids, W, b):
    ends = his_turn_end_ids.astype(jnp.int32).reshape(-1)
    out = _sc_call(encoder_output, ends, W, b)
    return out[0] + out[L]


# speedup vs baseline: 1.1476x; 1.0036x over previous
import functools

import jax
import jax.numpy as jnp
from jax import lax
from jax.experimental import pallas as pl
from jax.experimental.pallas import tpu as pltpu
from jax.experimental.pallas import tpu_sc as plsc

B, S, D, T = 16, 2048, 256, 32
NC, NS, L = 2, 16, 16
CHUNK = 128
HALF = S // 2
NCHUNK = HALF // CHUNK
ND = D // L
PAIRS = T * (T - 1) // 2
TPAD = T + L
PW = T * 4 * L


def _sload(ref, idx):
    return ref[pl.ds(idx, L)][0]


def _body(x_hbm, ends_hbm, w_hbm, b_hbm, out_hbm,
          buf0, buf1, buf2, ends_v, hi_v, lo_v, w2_v, b_v, uv_v, lrow_v,
          seg_v, p16_v, pab_v, lsum_v, seg_sh, loss_sh, sem0, sem1, sem2):
    c = lax.axis_index("c")
    s = lax.axis_index("s")
    bl = s // 2
    half = s % 2
    batch = c * 8 + bl
    t0 = half * HALF
    lanes = jnp.arange(L, dtype=jnp.int32)
    lane0 = lanes == 0
    zf = jnp.zeros((L,), jnp.float32)

    bufs = (buf0, buf1, buf2)
    sems = (sem0, sem1, sem2)
    NB = len(bufs)
    pending = [None] * NB
    for g0 in range(NB - 1):
        pending[g0] = pltpu.async_copy(
            x_hbm.at[batch, pl.ds(t0 + g0 * CHUNK, CHUNK)], bufs[g0], sems[g0])

    pltpu.sync_copy(ends_hbm.at[pl.ds(batch * T, T)], ends_v.at[pl.ds(0, T)])
    pltpu.sync_copy(w_hbm, w2_v)
    pltpu.sync_copy(b_hbm, b_v.at[pl.ds(0, 2)])

    onei = jnp.ones((L,), jnp.int32)
    hi0 = ends_v[pl.ds(0, L)] + onei
    hi1 = ends_v[pl.ds(L, L)] + onei
    hi_v[pl.ds(0, L)] = hi0
    hi_v[pl.ds(L, L)] = hi1
    lo_v[pl.ds(1, L)] = hi0
    lo_v[pl.ds(L + 1, L)] = hi1
    plsc.store_scatter(lo_v, [jnp.zeros((L,), jnp.int32)],
                       jnp.zeros((L,), jnp.int32), mask=(jnp.arange(L, dtype=jnp.int32) == 0))

    def _count_lt(ref, bound):
        n = jnp.zeros((L,), jnp.int32)
        onei = jnp.ones((L,), jnp.int32)
        zeroi = jnp.zeros((L,), jnp.int32)
        for gk in range(T // L):
            n = n + jnp.where(ref[pl.ds(gk * L, L)] < bound, onei, zeroi)
        return jnp.sum(n)

    @pl.loop(0, T)
    def _z(t):
        for dd in range(ND):
            seg_v[pl.ds(t * D + dd * L, L)] = zf

    for g in range(NCHUNK):
        pending[g % NB].wait()
        if g + NB - 1 < NCHUNK:
            pending[(g + NB - 1) % NB] = pltpu.async_copy(
                x_hbm.at[batch, pl.ds(t0 + (g + NB - 1) * CHUNK, CHUNK)],
                bufs[(g + NB - 1) % NB], sems[(g + NB - 1) % NB])
        base = t0 + g * CHUNK
        buf = bufs[g % NB]
        tlo = _count_lt(hi_v, base + 1)
        thi = _count_lt(lo_v, base + CHUNK)

        @pl.loop(tlo, thi)
        def _seg(t):
            lot = jnp.maximum(_sload(lo_v, t), base)
            hit = jnp.minimum(_sload(hi_v, t), base + CHUNK)

            @pl.when(hit > lot)
            def _acc():
                def sbody(srl, acc):
                    r = srl - base
                    return tuple(a + buf[r, pl.ds(dd * L, L)]
                                 for dd, a in enumerate(acc))

                accs = lax.fori_loop(lot, hit, sbody,
                                     tuple(zf for _ in range(ND)))
                for dd in range(ND):
                    seg_v[pl.ds(t * D + dd * L, L)] = (
                        seg_v[pl.ds(t * D + dd * L, L)] + accs[dd])

    @pl.loop(0, T)
    def _pz(t):
        for co in range(4):
            p16_v[pl.ds(t * (4 * L) + co * L, L)] = zf

    wlo = _count_lt(hi_v, t0 + 1)
    whi = _count_lt(lo_v, t0 + HALF)

    @pl.loop(wlo, whi)
    def _proj(t):
        sv = [seg_v[pl.ds(t * D + dd * L, L)] for dd in range(ND)]
        for co in range(4):
            wrow, woff = co % 2, (co // 2) * D
            acc = sv[0] * w2_v[wrow, pl.ds(woff, L)]
            for dd in range(1, ND):
                acc = acc + sv[dd] * w2_v[wrow, pl.ds(woff + dd * L, L)]
            p16_v[pl.ds(t * (4 * L) + co * L, L)] = acc

    pltpu.sync_copy(p16_v, seg_sh.at[pl.ds(s * PW, PW)])
    plsc.subcore_barrier()

    pltpu.sync_copy(seg_sh.at[pl.ds(bl * 2 * PW, 2 * PW)], pab_v)


    @pl.loop(0, T)
    def _uv(t):
        cnt = _sload(hi_v, t) - _sload(lo_v, t)
        invn = 1.0 / (zf + cnt.astype(jnp.float32))
        for co in range(4):
            v = (pab_v[pl.ds(t * (4 * L) + co * L, L)]
                 + pab_v[pl.ds(PW + t * (4 * L) + co * L, L)])
            val = (zf + jnp.sum(v)) * invn
            plsc.store_scatter(uv_v, [jnp.full((L,), co * TPAD, jnp.int32) + t],
                               val, mask=lane0)

    bvec = b_v[pl.ds(0, L)]
    b0 = bvec[0]
    b1 = bvec[1]

    def _cebody(j, acc):
        u0j = _sload(uv_v, j) + b0
        u1j = _sload(uv_v, TPAD + j) + b1
        tot = acc
        for gk in range(T // L):
            v0 = uv_v[pl.ds(2 * TPAD + gk * L, L)]
            v1 = uv_v[pl.ds(3 * TPAD + gk * L, L)]
            kv = lanes + (gk * L)
            l0 = v0 + u0j
            l1 = v1 + u1j
            m = jnp.maximum(l0, l1)
            e = jnp.exp(0.0 - jnp.abs(l0 - l1))
            r = e / (2.0 + e)
            r2 = r * r
            lny = 2.0 * r * (1.0 + r2 * (1.0 / 3.0 + r2 * (1.0 / 5.0 + r2 * (1.0 / 7.0 + r2 * (1.0 / 9.0)))))
            pick = jnp.where(kv == (j - 1), l1, l0) - (m + lny)
            tot = tot + jnp.where(kv < j, pick, jnp.zeros_like(pick))
        return tot

    jlo = half * (T // 2)
    lrow_v[...] = lax.fori_loop(jlo, jlo + T // 2, _cebody, zf)

    pltpu.sync_copy(lrow_v, loss_sh.at[pl.ds(s * L, L)])
    plsc.subcore_barrier()

    @pl.when(s == 0)
    def _final():
        pltpu.sync_copy(loss_sh, lsum_v)
        acc = lsum_v[pl.ds(0, L)]
        for se in range(1, NS):
            acc = acc + lsum_v[pl.ds(se * L, L)]
        total = jnp.sum(acc) * (-1.0 / (B * PAIRS))
        lrow_v[...] = jnp.zeros((L,), jnp.float32) + total
        pltpu.sync_copy(lrow_v, out_hbm.at[pl.ds(c * L, L)])


_sc_call = functools.partial(
    pl.kernel,
    out_type=jax.ShapeDtypeStruct((NC * L,), jnp.float32),
    mesh=plsc.VectorSubcoreMesh(core_axis_name="c", subcore_axis_name="s",
                                num_cores=NC, num_subcores=NS),
    compiler_params=pltpu.CompilerParams(use_tc_tiling_on_sc=True,
                                         needs_layout_passes=False),
    scratch_types=[
        pltpu.VMEM((CHUNK, D), jnp.float32),
        pltpu.VMEM((CHUNK, D), jnp.float32),
        pltpu.VMEM((CHUNK, D), jnp.float32),
        pltpu.VMEM((TPAD,), jnp.int32),
        pltpu.VMEM((TPAD,), jnp.int32),
        pltpu.VMEM((TPAD,), jnp.int32),
        pltpu.VMEM((2, 2 * D), jnp.float32),
        pltpu.VMEM((L,), jnp.float32),
        pltpu.VMEM((4 * TPAD,), jnp.float32),
        pltpu.VMEM((L,), jnp.float32),
        pltpu.VMEM((T * D,), jnp.float32),
        pltpu.VMEM((PW,), jnp.float32),
        pltpu.VMEM((2 * PW,), jnp.float32),
        pltpu.VMEM((NS * L,), jnp.float32),
        pltpu.VMEM_SHARED((NS * PW,), jnp.float32),
        pltpu.VMEM_SHARED((NS * L,), jnp.float32),
        pltpu.SemaphoreType.DMA,
        pltpu.SemaphoreType.DMA,
        pltpu.SemaphoreType.DMA,
    ],
)(_body)


def kernel(encoder_output, his_turn_end_ids, W, b):
    ends = his_turn_end_ids.astype(jnp.int32).reshape(-1)
    out = _sc_call(encoder_output, ends, W, b)
    return out[0] + out[L]
